# Initial kernel scaffold; baseline (speedup 1.0000x reference)
import jax
import jax.numpy as jnp
from jax.experimental import pallas as pl


def kernel(x, edge_index, W1, b1, W2, b2):
    raise NotImplementedError("write your pallas kernel here")


# speedup vs baseline: 16.0101x; 16.0101x over previous
import functools

import jax
import jax.numpy as jnp
from jax import lax
from jax.experimental import pallas as pl
from jax.experimental.pallas import tpu as pltpu
from jax.experimental.pallas import tpu_sc as plsc

B_EDGE = 128
N_WORKERS = 32
N_SUB = 16


def _sc_mesh():
    return plsc.VectorSubcoreMesh(core_axis_name="c", subcore_axis_name="s")


def _make_cnt_kernel(np_rows, cpt):
    rpt = np_rows // N_SUB

    @functools.partial(
        pl.kernel,
        mesh=_sc_mesh(),
        compiler_params=pltpu.CompilerParams(use_tc_tiling_on_sc=False),
        out_type=jax.ShapeDtypeStruct((2, np_rows, 16), jnp.float32),
        scratch_types=[
            pltpu.VMEM((cpt, B_EDGE), jnp.int32),
            pltpu.VMEM((B_EDGE, 16), jnp.float32),
            pltpu.VMEM_SHARED((np_rows, 16), jnp.float32),
        ],
    )
    def cnt_kernel(cols_hbm, ones_hbm, zeros_hbm, out_hbm, cols_v, ones_v, acc_s):
        c = lax.axis_index("c")
        s = lax.axis_index("s")
        wid = c * N_SUB + s
        pltpu.sync_copy(cols_hbm.at[wid], cols_v)
        pltpu.sync_copy(ones_hbm, ones_v)
        pltpu.sync_copy(zeros_hbm.at[pl.ds(s * rpt, rpt)],
                        acc_s.at[pl.ds(s * rpt, rpt)])
        plsc.subcore_barrier()

        def body(j, carry):
            pltpu.sync_copy(ones_v, acc_s.at[cols_v.at[j]], add=True)
            return carry

        lax.fori_loop(0, cpt, body, 0)
        plsc.subcore_barrier()
        pltpu.sync_copy(acc_s.at[pl.ds(s * rpt, rpt)],
                        out_hbm.at[c, pl.ds(s * rpt, rpt)])

    return cnt_kernel


def _make_agg_kernel(np_rows, cpt, d, n_table):
    rpt = np_rows // N_SUB
    params = (None if d % 128 == 0
              else pltpu.CompilerParams(use_tc_tiling_on_sc=False))

    @functools.partial(
        pl.kernel,
        mesh=_sc_mesh(),
        compiler_params=params,
        out_type=jax.ShapeDtypeStruct((2, np_rows, d), jnp.float32),
        scratch_types=[
            pltpu.VMEM((cpt, B_EDGE), jnp.int32),
            pltpu.VMEM((cpt, B_EDGE), jnp.int32),
            pltpu.VMEM((B_EDGE, d), jnp.float32),
            pltpu.VMEM_SHARED((np_rows, d), jnp.float32),
            pltpu.SemaphoreType.DMA,
        ],
    )
    def agg_kernel(y_hbm, rows_hbm, cols_hbm, zeros_hbm, out_hbm,
                   rows_v, cols_v, buf_v, acc_s, sem):
        c = lax.axis_index("c")
        s = lax.axis_index("s")
        wid = c * N_SUB + s
        pltpu.sync_copy(rows_hbm.at[wid], rows_v)
        pltpu.sync_copy(cols_hbm.at[wid], cols_v)
        pltpu.sync_copy(zeros_hbm.at[pl.ds(s * rpt, rpt)],
                        acc_s.at[pl.ds(s * rpt, rpt)])
        plsc.subcore_barrier()

        def body(j, carry):
            pltpu.async_copy(y_hbm.at[rows_v.at[j]], buf_v, sem).wait()
            pltpu.sync_copy(buf_v, acc_s.at[cols_v.at[j]], add=True)
            return carry

        lax.fori_loop(0, cpt, body, 0)
        plsc.subcore_barrier()
        pltpu.sync_copy(acc_s.at[pl.ds(s * rpt, rpt)],
                        out_hbm.at[c, pl.ds(s * rpt, rpt)])

    return agg_kernel


def _tc_scale_kernel(n, d_in, hid):
    def body(x_ref, w1_ref, cnt_ref, y1_ref, dinv_ref):
        cnt = cnt_ref[0, :n, 0:1] + cnt_ref[1, :n, 0:1]
        dinv = lax.rsqrt(cnt + 1.0)
        xw = jnp.dot(x_ref[...], w1_ref[...],
                     preferred_element_type=jnp.float32)
        y1_ref[...] = xw * dinv
        dinv_ref[...] = dinv

    return pl.pallas_call(
        body,
        out_shape=(
            jax.ShapeDtypeStruct((n, hid), jnp.float32),
            jax.ShapeDtypeStruct((n, 1), jnp.float32),
        ),
    )


def _tc_mid_kernel(n, hid, out_d):
    def body(acc_ref, y1_ref, dinv_ref, mask2_ref, w2_ref, b1_ref, y2_ref):
        a = acc_ref[0, :n, :] + acc_ref[1, :n, :] + y1_ref[...]
        g = a * dinv_ref[...] + b1_ref[...]
        h = jnp.maximum(g, 0.0) * mask2_ref[...]
        t = jnp.dot(h, w2_ref[...], preferred_element_type=jnp.float32)
        y2_ref[...] = t * dinv_ref[...]

    return pl.pallas_call(
        body,
        out_shape=jax.ShapeDtypeStruct((n, out_d), jnp.float32),
    )


def _tc_out_kernel(n, out_d):
    def body(acc_ref, y2_ref, dinv_ref, b2_ref, out_ref):
        a = acc_ref[0, :n, :] + acc_ref[1, :n, :] + y2_ref[...]
        out_ref[...] = a * dinv_ref[...] + b2_ref[...]

    return pl.pallas_call(
        body,
        out_shape=jax.ShapeDtypeStruct((n, out_d), jnp.float32),
    )


def kernel(x, edge_index, W1, b1, W2, b2):
    n, d_in = x.shape
    hid = W1.shape[1]
    out_d = W2.shape[1]
    e = edge_index.shape[1]

    np_rows = -(-(n + 1) // (N_SUB * 8)) * (N_SUB * 8)
    cpt = -(-e // (N_WORKERS * B_EDGE))
    e_pad = N_WORKERS * cpt * B_EDGE

    rows = edge_index[0].astype(jnp.int32)
    cols = edge_index[1].astype(jnp.int32)
    pad = e_pad - e
    rows_p = jnp.concatenate(
        [rows, jnp.zeros((pad,), jnp.int32)]).reshape(N_WORKERS, cpt, B_EDGE)
    cols_p = jnp.concatenate(
        [cols, jnp.full((pad,), n, jnp.int32)]).reshape(N_WORKERS, cpt, B_EDGE)

    ones16 = jnp.zeros((B_EDGE, 16), jnp.float32).at[:, 0].set(1.0)
    z16 = jnp.zeros((np_rows, 16), jnp.float32)
    z_hid = jnp.zeros((np_rows, hid), jnp.float32)
    z_out = jnp.zeros((np_rows, out_d), jnp.float32)

    mask2 = jnp.where(
        jax.random.bernoulli(jax.random.key(42), 0.5, (n, hid)), 2.0, 0.0
    ).astype(jnp.float32)

    cnt = _make_cnt_kernel(np_rows, cpt)(cols_p, ones16, z16)
    y1, dinv = _tc_scale_kernel(n, d_in, hid)(x, W1, cnt)
    acc1 = _make_agg_kernel(np_rows, cpt, hid, n)(y1, rows_p, cols_p, z_hid)
    y2 = _tc_mid_kernel(n, hid, out_d)(
        acc1, y1, dinv, mask2, W2, b1.reshape(1, hid))
    acc2 = _make_agg_kernel(np_rows, cpt, out_d, n)(y2, rows_p, cols_p, z_out)
    out = _tc_out_kernel(n, out_d)(acc2, y2, dinv, b2.reshape(1, out_d))
    return out
